# initial kernel scaffold (unmeasured)
import jax
import jax.numpy as jnp
from jax import lax
from jax.experimental import pallas as pl
from jax.experimental.pallas import tpu as pltpu

N_DEV = 4
B, SQ, DM = 4, 256, 1024
HG, HL, DH = 32, 8, 128
NQB, QBLK = 4, 64
NT = 16
KSEL = NT * QBLK
SCALE = 0.08838834764831843


def kernel(x, Wq, K_ext, V_ext, Wo):
    K_r = K_ext.reshape(B, NT, NQB, QBLK, HG, DH)
    V_r = V_ext.reshape(B, NT, NQB, QBLK, HG, DH)

    def body(x_ref, wq_ref, k_hbm, v_hbm, wo_ref, out_ref,
             x_all, wq_bf, wo_bf, q_bf, kst, vst, acc, rs_buf,
             ag_send, ag_recv, rs_send, rs_recv, k_sems, v_sems):
        my = lax.axis_index("i")
        right = lax.rem(my + 1, N_DEV)
        left = lax.rem(my + N_DEV - 1, N_DEV)
        h0 = my * HL

        barrier = pltpu.get_barrier_semaphore()
        for nbr in (left, right):
            pl.semaphore_signal(barrier, inc=1, device_id=(nbr,),
                                device_id_type=pl.DeviceIdType.MESH)
        pl.semaphore_wait(barrier, 2)

        wq_bf[...] = wq_ref[...].astype(jnp.bfloat16)
        wo_bf[...] = wo_ref[...].astype(jnp.bfloat16)
        x_all[pl.ds(my, 1)] = x_ref[...].astype(jnp.bfloat16)

        for h in range(N_DEV - 1):
            c = lax.rem(my + (N_DEV - h), N_DEV)
            rdma = pltpu.make_async_remote_copy(
                src_ref=x_all.at[pl.ds(c, 1)],
                dst_ref=x_all.at[pl.ds(c, 1)],
                send_sem=ag_send.at[h], recv_sem=ag_recv.at[h],
                device_id=(right,), device_id_type=pl.DeviceIdType.MESH)
            rdma.start()
            rdma.wait()

        for b in range(B):
            xb = x_all[b]
            for h in range(HL):
                q = jnp.dot(xb, wq_bf[:, h * DH:(h + 1) * DH],
                            preferred_element_type=jnp.float32)
                q_bf[h] = (q * SCALE).astype(jnp.bfloat16)

            for qb in range(NQB):
                copies = []
                for h in range(HL):
                    ck = pltpu.make_async_copy(
                        k_hbm.at[b, :, qb, :, h0 + h, :], kst.at[h],
                        k_sems.at[h])
                    cv = pltpu.make_async_copy(
                        v_hbm.at[b, :, qb, :, h0 + h, :], vst.at[h],
                        v_sems.at[h])
                    ck.start()
                    cv.start()
                    copies += [ck, cv]
                for cpy in copies:
                    cpy.wait()

                psum = None
                for h in range(HL):
                    kh = kst[h].reshape(KSEL, DH).astype(jnp.bfloat16)
                    vh = vst[h].reshape(KSEL, DH).astype(jnp.bfloat16)
                    qh = q_bf[h, qb * QBLK:(qb + 1) * QBLK, :]
                    s = lax.dot_general(qh, kh, (((1,), (1,)), ((), ())),
                                        preferred_element_type=jnp.float32)
                    m = jnp.max(s, axis=-1, keepdims=True)
                    e = jnp.exp(s - m)
                    w = (e / jnp.sum(e, axis=-1, keepdims=True)
                         ).astype(jnp.bfloat16)
                    o = jnp.dot(w, vh, preferred_element_type=jnp.float32)
                    p = jnp.dot(o.astype(jnp.bfloat16),
                                wo_bf[h * DH:(h + 1) * DH, :],
                                preferred_element_type=jnp.float32)
                    psum = p if psum is None else psum + p
                acc[b, qb * QBLK:(qb + 1) * QBLK, :] = psum

        for s in range(N_DEV - 1):
            c_send = lax.rem(my + (N_DEV - 1 - s), N_DEV)
            rdma = pltpu.make_async_remote_copy(
                src_ref=acc.at[pl.ds(c_send, 1)],
                dst_ref=rs_buf.at[pl.ds(s, 1)],
                send_sem=rs_send.at[s], recv_sem=rs_recv.at[s],
                device_id=(right,), device_id_type=pl.DeviceIdType.MESH)
            rdma.start()
            rdma.wait()
            c_add = lax.rem(my + (N_DEV - 2 - s), N_DEV)
            acc[pl.ds(c_add, 1)] = acc[pl.ds(c_add, 1)] + rs_buf[pl.ds(s, 1)]

        out_ref[...] = acc[pl.ds(my, 1)]

    return pl.pallas_call(
        body,
        out_shape=jax.ShapeDtypeStruct((1, SQ, DM), jnp.float32),
        in_specs=[
            pl.BlockSpec(memory_space=pltpu.VMEM),
            pl.BlockSpec(memory_space=pltpu.VMEM),
            pl.BlockSpec(memory_space=pltpu.ANY),
            pl.BlockSpec(memory_space=pltpu.ANY),
            pl.BlockSpec(memory_space=pltpu.VMEM),
        ],
        out_specs=pl.BlockSpec(memory_space=pltpu.VMEM),
        scratch_shapes=[
            pltpu.VMEM((B, SQ, DM), jnp.bfloat16),
            pltpu.VMEM((DM, DM), jnp.bfloat16),
            pltpu.VMEM((DM, DM), jnp.bfloat16),
            pltpu.VMEM((HL, SQ, DH), jnp.bfloat16),
            pltpu.VMEM((HL, NT, QBLK, DH), jnp.float32),
            pltpu.VMEM((HL, NT, QBLK, DH), jnp.float32),
            pltpu.VMEM((B, SQ, DM), jnp.float32),
            pltpu.VMEM((N_DEV - 1, SQ, DM), jnp.float32),
            pltpu.SemaphoreType.DMA((N_DEV - 1,)),
            pltpu.SemaphoreType.DMA((N_DEV - 1,)),
            pltpu.SemaphoreType.DMA((N_DEV - 1,)),
            pltpu.SemaphoreType.DMA((N_DEV - 1,)),
            pltpu.SemaphoreType.DMA((HL,)),
            pltpu.SemaphoreType.DMA((HL,)),
        ],
        compiler_params=pltpu.CompilerParams(collective_id=0),
    )(x, Wq, K_r, V_r, Wo)


# baseline (device time: 211356 ns/iter reference)
import jax
import jax.numpy as jnp
from jax import lax
from jax.experimental import pallas as pl
from jax.experimental.pallas import tpu as pltpu

N_DEV = 4
B, SQ, DM = 4, 256, 1024
HG, HL, DH = 32, 8, 128
NQB, QBLK = 4, 64
NT = 16
KSEL = NT * QBLK
SCALE = 0.08838834764831843


def kernel(x, Wq, K_ext, V_ext, Wo):
    K_r = K_ext.reshape(B, NT, NQB, QBLK, HG, DH)
    V_r = V_ext.reshape(B, NT, NQB, QBLK, HG, DH)

    def body(x_ref, wq_ref, k_hbm, v_hbm, wo_ref, out_ref,
             x_all, wq_bf, wo_bf, q_bf, kst, vst, acc, rs_buf,
             ag_send, ag_recv, rs_send, rs_recv, k_sems, v_sems):
        my = lax.axis_index("i")
        right = lax.rem(my + 1, N_DEV)
        left = lax.rem(my + N_DEV - 1, N_DEV)
        h0 = my * HL

        barrier = pltpu.get_barrier_semaphore()
        for nbr in (left, right):
            pl.semaphore_signal(barrier, inc=1, device_id=(nbr,),
                                device_id_type=pl.DeviceIdType.MESH)
        pl.semaphore_wait(barrier, 2)

        wq_bf[...] = wq_ref[...].astype(jnp.bfloat16)
        wo_bf[...] = wo_ref[...].astype(jnp.bfloat16)
        x_all[pl.ds(my, 1)] = x_ref[...].astype(jnp.bfloat16)

        for h in range(N_DEV - 1):
            c = lax.rem(my + (N_DEV - h), N_DEV)
            rdma = pltpu.make_async_remote_copy(
                src_ref=x_all.at[pl.ds(c, 1)],
                dst_ref=x_all.at[pl.ds(c, 1)],
                send_sem=ag_send.at[h], recv_sem=ag_recv.at[h],
                device_id=(right,), device_id_type=pl.DeviceIdType.MESH)
            rdma.start()
            rdma.wait()

        for b in range(B):
            xb = x_all[b]
            for h in range(HL):
                q = jnp.dot(xb, wq_bf[:, h * DH:(h + 1) * DH],
                            preferred_element_type=jnp.float32)
                q_bf[h] = (q * SCALE).astype(jnp.bfloat16)

            for qb in range(NQB):
                copies = []
                for h in range(HL):
                    ck = pltpu.make_async_copy(
                        k_hbm.at[b, :, qb, :, h0 + h, :], kst.at[h],
                        k_sems.at[h])
                    cv = pltpu.make_async_copy(
                        v_hbm.at[b, :, qb, :, h0 + h, :], vst.at[h],
                        v_sems.at[h])
                    ck.start()
                    cv.start()
                    copies += [ck, cv]
                for cpy in copies:
                    cpy.wait()

                psum = None
                for h in range(HL):
                    kh = kst[h].reshape(KSEL, DH).astype(jnp.bfloat16)
                    vh = vst[h].reshape(KSEL, DH).astype(jnp.bfloat16)
                    qh = q_bf[h, qb * QBLK:(qb + 1) * QBLK, :]
                    s = lax.dot_general(qh, kh, (((1,), (1,)), ((), ())),
                                        preferred_element_type=jnp.float32)
                    m = jnp.max(s, axis=-1, keepdims=True)
                    e = jnp.exp(s - m)
                    w = (e / jnp.sum(e, axis=-1, keepdims=True)
                         ).astype(jnp.bfloat16)
                    o = jnp.dot(w, vh, preferred_element_type=jnp.float32)
                    p = jnp.dot(o.astype(jnp.bfloat16),
                                wo_bf[h * DH:(h + 1) * DH, :],
                                preferred_element_type=jnp.float32)
                    psum = p if psum is None else psum + p
                acc[b, qb * QBLK:(qb + 1) * QBLK, :] = psum

        for s in range(N_DEV - 1):
            c_send = lax.rem(my + (N_DEV - 1 - s), N_DEV)
            rdma = pltpu.make_async_remote_copy(
                src_ref=acc.at[pl.ds(c_send, 1)],
                dst_ref=rs_buf.at[pl.ds(s, 1)],
                send_sem=rs_send.at[s], recv_sem=rs_recv.at[s],
                device_id=(right,), device_id_type=pl.DeviceIdType.MESH)
            rdma.start()
            rdma.wait()
            c_add = lax.rem(my + (N_DEV - 2 - s), N_DEV)
            acc[pl.ds(c_add, 1)] = acc[pl.ds(c_add, 1)] + rs_buf[pl.ds(s, 1)]

        out_ref[...] = acc[pl.ds(my, 1)]

    return pl.pallas_call(
        body,
        out_shape=jax.ShapeDtypeStruct((1, SQ, DM), jnp.float32),
        in_specs=[
            pl.BlockSpec(memory_space=pltpu.VMEM),
            pl.BlockSpec(memory_space=pltpu.VMEM),
            pl.BlockSpec(memory_space=pl.ANY),
            pl.BlockSpec(memory_space=pl.ANY),
            pl.BlockSpec(memory_space=pltpu.VMEM),
        ],
        out_specs=pl.BlockSpec(memory_space=pltpu.VMEM),
        scratch_shapes=[
            pltpu.VMEM((B, SQ, DM), jnp.bfloat16),
            pltpu.VMEM((DM, DM), jnp.bfloat16),
            pltpu.VMEM((DM, DM), jnp.bfloat16),
            pltpu.VMEM((HL, SQ, DH), jnp.bfloat16),
            pltpu.VMEM((HL, NT, QBLK, DH), jnp.float32),
            pltpu.VMEM((HL, NT, QBLK, DH), jnp.float32),
            pltpu.VMEM((B, SQ, DM), jnp.float32),
            pltpu.VMEM((N_DEV - 1, SQ, DM), jnp.float32),
            pltpu.SemaphoreType.DMA((N_DEV - 1,)),
            pltpu.SemaphoreType.DMA((N_DEV - 1,)),
            pltpu.SemaphoreType.DMA((N_DEV - 1,)),
            pltpu.SemaphoreType.DMA((N_DEV - 1,)),
            pltpu.SemaphoreType.DMA((HL,)),
            pltpu.SemaphoreType.DMA((HL,)),
        ],
        compiler_params=pltpu.CompilerParams(collective_id=0),
    )(x, Wq, K_r, V_r, Wo)


# device time: 132637 ns/iter; 1.5935x vs baseline; 1.5935x over previous
import jax
import jax.numpy as jnp
from jax import lax
from jax.experimental import pallas as pl
from jax.experimental.pallas import tpu as pltpu

N_DEV = 4
B, SQ, DM = 4, 256, 1024
HG, HL, DH = 32, 8, 128
NQB, QBLK = 4, 64
NT = 16
KSEL = NT * QBLK
SCALE = 0.08838834764831843


def kernel(x, Wq, K_ext, V_ext, Wo):
    K_r = K_ext.reshape(B, NT, NQB, QBLK, HG, DH)
    V_r = V_ext.reshape(B, NT, NQB, QBLK, HG, DH)

    def body(x_ref, wq_ref, k_hbm, v_hbm, wo_ref, out_ref,
             x_all, wq_bf, wo_bf, q_bf, kst, vst, k_bf, v_bf, acc, rs_buf,
             ag_send, ag_recv, rs_send, rs_recv, k_sems, v_sems):
        my = lax.axis_index("i")
        right = lax.rem(my + 1, N_DEV)
        left = lax.rem(my + N_DEV - 1, N_DEV)
        h0 = my * HL

        def batch_of(j):
            return lax.rem(my + (N_DEV - j), N_DEV)

        barrier = pltpu.get_barrier_semaphore()
        for nbr in (left, right):
            pl.semaphore_signal(barrier, inc=1, device_id=(nbr,),
                                device_id_type=pl.DeviceIdType.MESH)
        pl.semaphore_wait(barrier, 2)

        x_all[pl.ds(my, 1)] = x_ref[...].astype(jnp.bfloat16)

        def ag_hop(h):
            c = lax.rem(my + (N_DEV - h), N_DEV)
            r = pltpu.make_async_remote_copy(
                src_ref=x_all.at[pl.ds(c, 1)],
                dst_ref=x_all.at[pl.ds(c, 1)],
                send_sem=ag_send.at[h], recv_sem=ag_recv.at[h],
                device_id=(right,), device_id_type=pl.DeviceIdType.MESH)
            r.start()
            return r

        ag = [ag_hop(0)]

        def issue_stage(t):
            j, qb = divmod(t, NQB)
            bb = batch_of(j)
            slot = t % 2
            ck = pltpu.make_async_copy(
                k_hbm.at[bb, :, qb, :, pl.ds(h0, HL), :], kst.at[slot],
                k_sems.at[slot])
            cv = pltpu.make_async_copy(
                v_hbm.at[bb, :, qb, :, pl.ds(h0, HL), :], vst.at[slot],
                v_sems.at[slot])
            ck.start()
            cv.start()
            return (ck, cv)

        desc = {0: issue_stage(0), 1: issue_stage(1)}

        wq_bf[...] = wq_ref[...].astype(jnp.bfloat16)
        wo_bf[...] = wo_ref[...].astype(jnp.bfloat16)

        rs = [None] * (N_DEV - 1)

        def rs_step_start(s):
            c_send = lax.rem(my + (N_DEV - 1 - s), N_DEV)
            r = pltpu.make_async_remote_copy(
                src_ref=acc.at[pl.ds(c_send, 1)],
                dst_ref=rs_buf.at[pl.ds(s, 1)],
                send_sem=rs_send.at[s], recv_sem=rs_recv.at[s],
                device_id=(right,), device_id_type=pl.DeviceIdType.MESH)
            r.start()
            rs[s] = r

        def rs_step_finish(s):
            rs[s].wait()
            c_add = lax.rem(my + (N_DEV - 2 - s), N_DEV)
            acc[pl.ds(c_add, 1)] = acc[pl.ds(c_add, 1)] + rs_buf[pl.ds(s, 1)]

        def compute_batch(j):
            bb = batch_of(j)
            xb = x_all[pl.ds(bb, 1)][0]
            q = jnp.dot(xb, wq_bf[...],
                        preferred_element_type=jnp.float32)
            q_bf[...] = (q * SCALE).astype(jnp.bfloat16)

            for qb in range(NQB):
                t = j * NQB + qb
                slot = t % 2
                ck, cv = desc.pop(t)
                ck.wait()
                cv.wait()
                k_bf[...] = kst[slot].reshape(KSEL, HL * DH
                                              ).astype(jnp.bfloat16)
                v_bf[...] = vst[slot].reshape(KSEL, HL * DH
                                              ).astype(jnp.bfloat16)
                if t + 2 < N_DEV * NQB:
                    desc[t + 2] = issue_stage(t + 2)

                psum = None
                for h in range(HL):
                    kh = k_bf[:, h * DH:(h + 1) * DH]
                    vh = v_bf[:, h * DH:(h + 1) * DH]
                    qh = q_bf[qb * QBLK:(qb + 1) * QBLK,
                              h * DH:(h + 1) * DH]
                    s = lax.dot_general(qh, kh, (((1,), (1,)), ((), ())),
                                        preferred_element_type=jnp.float32)
                    m = jnp.max(s, axis=-1, keepdims=True)
                    e = jnp.exp(s - m)
                    w = (e / jnp.sum(e, axis=-1, keepdims=True)
                         ).astype(jnp.bfloat16)
                    o = jnp.dot(w, vh, preferred_element_type=jnp.float32)
                    p = jnp.dot(o.astype(jnp.bfloat16),
                                wo_bf[h * DH:(h + 1) * DH, :],
                                preferred_element_type=jnp.float32)
                    psum = p if psum is None else psum + p
                acc[pl.ds(bb, 1), qb * QBLK:(qb + 1) * QBLK, :] = psum[None]

        compute_batch(0)

        for j in range(1, N_DEV):
            ag[j - 1].wait()
            if j < N_DEV - 1:
                ag.append(ag_hop(j))
            compute_batch(j)
            if j >= 2:
                rs_step_finish(j - 2)
            rs_step_start(j - 1)

        rs_step_finish(N_DEV - 2)
        out_ref[...] = acc[pl.ds(my, 1)]

    return pl.pallas_call(
        body,
        out_shape=jax.ShapeDtypeStruct((1, SQ, DM), jnp.float32),
        in_specs=[
            pl.BlockSpec(memory_space=pltpu.MemorySpace.VMEM),
            pl.BlockSpec(memory_space=pltpu.MemorySpace.VMEM),
            pl.BlockSpec(memory_space=pl.ANY),
            pl.BlockSpec(memory_space=pl.ANY),
            pl.BlockSpec(memory_space=pltpu.MemorySpace.VMEM),
        ],
        out_specs=pl.BlockSpec(memory_space=pltpu.MemorySpace.VMEM),
        scratch_shapes=[
            pltpu.VMEM((B, SQ, DM), jnp.bfloat16),
            pltpu.VMEM((DM, DM), jnp.bfloat16),
            pltpu.VMEM((DM, DM), jnp.bfloat16),
            pltpu.VMEM((SQ, HL * DH), jnp.bfloat16),
            pltpu.VMEM((2, NT, QBLK, HL, DH), jnp.float32),
            pltpu.VMEM((2, NT, QBLK, HL, DH), jnp.float32),
            pltpu.VMEM((KSEL, HL * DH), jnp.bfloat16),
            pltpu.VMEM((KSEL, HL * DH), jnp.bfloat16),
            pltpu.VMEM((B, SQ, DM), jnp.float32),
            pltpu.VMEM((N_DEV - 1, SQ, DM), jnp.float32),
            pltpu.SemaphoreType.DMA((N_DEV - 1,)),
            pltpu.SemaphoreType.DMA((N_DEV - 1,)),
            pltpu.SemaphoreType.DMA((N_DEV - 1,)),
            pltpu.SemaphoreType.DMA((N_DEV - 1,)),
            pltpu.SemaphoreType.DMA((2,)),
            pltpu.SemaphoreType.DMA((2,)),
        ],
        compiler_params=pltpu.CompilerParams(
            collective_id=0, vmem_limit_bytes=64 * 1024 * 1024),
    )(x, Wq, K_r, V_r, Wo)


# device time: 94117 ns/iter; 2.2457x vs baseline; 1.4093x over previous
import jax
import jax.numpy as jnp
from jax import lax
from jax.experimental import pallas as pl
from jax.experimental.pallas import tpu as pltpu

N_DEV = 4
B, SQ, DM = 4, 256, 1024
HG, HL, DH = 32, 8, 128
NQB, QBLK = 4, 64
NT = 16
KSEL = NT * QBLK
SCALE = 0.08838834764831843


def kernel(x, Wq, K_ext, V_ext, Wo):
    K_r = K_ext.reshape(B, NT, NQB, QBLK, HG, DH)
    V_r = V_ext.reshape(B, NT, NQB, QBLK, HG, DH)

    def body(x_ref, wq_ref, k_hbm, v_hbm, wo_ref, out_ref,
             x_all, wq_bf, wo_bf, q_bf, kst, vst, k_bf, v_bf, ctx_bf,
             acc, rs_buf,
             ag_send, ag_recv, rs_send, rs_recv, k_sems, v_sems):
        my = lax.axis_index("i")
        right = lax.rem(my + 1, N_DEV)
        left = lax.rem(my + N_DEV - 1, N_DEV)
        h0 = my * HL

        def batch_of(j):
            return lax.rem(my + (N_DEV - j), N_DEV)

        barrier = pltpu.get_barrier_semaphore()
        for nbr in (left, right):
            pl.semaphore_signal(barrier, inc=1, device_id=(nbr,),
                                device_id_type=pl.DeviceIdType.MESH)
        pl.semaphore_wait(barrier, 2)

        x_all[pl.ds(my, 1)] = x_ref[...].astype(jnp.bfloat16)

        def ag_hop(h):
            c = lax.rem(my + (N_DEV - h), N_DEV)
            r = pltpu.make_async_remote_copy(
                src_ref=x_all.at[pl.ds(c, 1)],
                dst_ref=x_all.at[pl.ds(c, 1)],
                send_sem=ag_send.at[h], recv_sem=ag_recv.at[h],
                device_id=(right,), device_id_type=pl.DeviceIdType.MESH)
            r.start()
            return r

        ag = [ag_hop(0)]

        def issue_stage(t):
            j, qb = divmod(t, NQB)
            bb = batch_of(j)
            slot = t % 2
            ck = pltpu.make_async_copy(
                k_hbm.at[bb, :, qb, :, pl.ds(h0, HL), :], kst.at[slot],
                k_sems.at[slot])
            cv = pltpu.make_async_copy(
                v_hbm.at[bb, :, qb, :, pl.ds(h0, HL), :], vst.at[slot],
                v_sems.at[slot])
            ck.start()
            cv.start()
            return (ck, cv)

        desc = {0: issue_stage(0), 1: issue_stage(1)}

        wq_bf[...] = wq_ref[...].astype(jnp.bfloat16)
        wo_bf[...] = wo_ref[...].astype(jnp.bfloat16)

        rs = [None] * (N_DEV - 1)

        def rs_step_start(s):
            c_send = lax.rem(my + (N_DEV - 1 - s), N_DEV)
            r = pltpu.make_async_remote_copy(
                src_ref=acc.at[pl.ds(c_send, 1)],
                dst_ref=rs_buf.at[pl.ds(s, 1)],
                send_sem=rs_send.at[s], recv_sem=rs_recv.at[s],
                device_id=(right,), device_id_type=pl.DeviceIdType.MESH)
            r.start()
            rs[s] = r

        def rs_step_finish(s):
            rs[s].wait()
            c_add = lax.rem(my + (N_DEV - 2 - s), N_DEV)
            acc[pl.ds(c_add, 1)] = acc[pl.ds(c_add, 1)] + rs_buf[pl.ds(s, 1)]

        def compute_batch(j):
            bb = batch_of(j)
            xb = x_all[pl.ds(bb, 1)][0]
            q = jnp.dot(xb, wq_bf[...],
                        preferred_element_type=jnp.float32)
            q_bf[...] = (q * SCALE).astype(jnp.bfloat16)

            for qb in range(NQB):
                t = j * NQB + qb
                slot = t % 2
                ck, cv = desc.pop(t)
                ck.wait()
                cv.wait()
                k_bf[...] = kst[slot].reshape(KSEL, HL * DH
                                              ).astype(jnp.bfloat16)
                v_bf[...] = vst[slot].reshape(KSEL, HL * DH
                                              ).astype(jnp.bfloat16)
                if t + 2 < N_DEV * NQB:
                    desc[t + 2] = issue_stage(t + 2)

                for h in range(HL):
                    kh = k_bf[:, h * DH:(h + 1) * DH]
                    vh = v_bf[:, h * DH:(h + 1) * DH]
                    qh = q_bf[qb * QBLK:(qb + 1) * QBLK,
                              h * DH:(h + 1) * DH]
                    s = lax.dot_general(qh, kh, (((1,), (1,)), ((), ())),
                                        preferred_element_type=jnp.float32)
                    e = jnp.exp(s)
                    inv = 1.0 / jnp.sum(e, axis=-1, keepdims=True)
                    o = jnp.dot(e.astype(jnp.bfloat16), vh,
                                preferred_element_type=jnp.float32)
                    ctx_bf[:, h * DH:(h + 1) * DH] = (o * inv
                                                      ).astype(jnp.bfloat16)
                psum = jnp.dot(ctx_bf[...], wo_bf[...],
                               preferred_element_type=jnp.float32)
                acc[pl.ds(bb, 1), qb * QBLK:(qb + 1) * QBLK, :] = psum[None]

        compute_batch(0)

        for j in range(1, N_DEV):
            ag[j - 1].wait()
            if j < N_DEV - 1:
                ag.append(ag_hop(j))
            compute_batch(j)
            if j >= 2:
                rs_step_finish(j - 2)
            rs_step_start(j - 1)

        rs_step_finish(N_DEV - 2)
        out_ref[...] = acc[pl.ds(my, 1)]

    return pl.pallas_call(
        body,
        out_shape=jax.ShapeDtypeStruct((1, SQ, DM), jnp.float32),
        in_specs=[
            pl.BlockSpec(memory_space=pltpu.MemorySpace.VMEM),
            pl.BlockSpec(memory_space=pltpu.MemorySpace.VMEM),
            pl.BlockSpec(memory_space=pl.ANY),
            pl.BlockSpec(memory_space=pl.ANY),
            pl.BlockSpec(memory_space=pltpu.MemorySpace.VMEM),
        ],
        out_specs=pl.BlockSpec(memory_space=pltpu.MemorySpace.VMEM),
        scratch_shapes=[
            pltpu.VMEM((B, SQ, DM), jnp.bfloat16),
            pltpu.VMEM((DM, DM), jnp.bfloat16),
            pltpu.VMEM((DM, DM), jnp.bfloat16),
            pltpu.VMEM((SQ, HL * DH), jnp.bfloat16),
            pltpu.VMEM((2, NT, QBLK, HL, DH), jnp.float32),
            pltpu.VMEM((2, NT, QBLK, HL, DH), jnp.float32),
            pltpu.VMEM((KSEL, HL * DH), jnp.bfloat16),
            pltpu.VMEM((KSEL, HL * DH), jnp.bfloat16),
            pltpu.VMEM((QBLK, HL * DH), jnp.bfloat16),
            pltpu.VMEM((B, SQ, DM), jnp.float32),
            pltpu.VMEM((N_DEV - 1, SQ, DM), jnp.float32),
            pltpu.SemaphoreType.DMA((N_DEV - 1,)),
            pltpu.SemaphoreType.DMA((N_DEV - 1,)),
            pltpu.SemaphoreType.DMA((N_DEV - 1,)),
            pltpu.SemaphoreType.DMA((N_DEV - 1,)),
            pltpu.SemaphoreType.DMA((2,)),
            pltpu.SemaphoreType.DMA((2,)),
        ],
        compiler_params=pltpu.CompilerParams(
            collective_id=0, vmem_limit_bytes=64 * 1024 * 1024),
    )(x, Wq, K_r, V_r, Wo)


# device time: 72721 ns/iter; 2.9064x vs baseline; 1.2942x over previous
import os

import jax
import jax.numpy as jnp
from jax import lax
from jax.experimental import pallas as pl
from jax.experimental.pallas import tpu as pltpu

N_DEV = 4
B, SQ, DM = 4, 256, 1024
HG, HL, DH = 32, 8, 128
NQB, QBLK = 4, 64
NT = 16
KSEL = NT * QBLK
SCALE = 0.08838834764831843
ABLATE = os.environ.get("ABLATE", "")


def kernel(x, Wq, K_ext, V_ext, Wo):
    K_r = K_ext.reshape(B, NT, NQB, QBLK, HG, DH)
    V_r = V_ext.reshape(B, NT, NQB, QBLK, HG, DH)

    def body(x_ref, wq_ref, k_hbm, v_hbm, wo_ref, out_ref,
             x_all, wq_bf, wo_bf, q_bf, kst, vst, k_bf, v_bf, ctx_bf,
             acc, rs_buf,
             ag_send, ag_recv, rs_send, rs_recv, k_sems, v_sems):
        my = lax.axis_index("i")
        right = lax.rem(my + 1, N_DEV)
        left = lax.rem(my + N_DEV - 1, N_DEV)
        h0 = my * HL

        def batch_of(j):
            return lax.rem(my + (N_DEV - j), N_DEV)

        barrier = pltpu.get_barrier_semaphore()
        for nbr in (left, right):
            pl.semaphore_signal(barrier, inc=1, device_id=(nbr,),
                                device_id_type=pl.DeviceIdType.MESH)
        pl.semaphore_wait(barrier, 2)

        x_all[pl.ds(my, 1)] = x_ref[...].astype(jnp.bfloat16)

        def ag_hop(h):
            c = lax.rem(my + (N_DEV - h), N_DEV)
            r = pltpu.make_async_remote_copy(
                src_ref=x_all.at[pl.ds(c, 1)],
                dst_ref=x_all.at[pl.ds(c, 1)],
                send_sem=ag_send.at[h], recv_sem=ag_recv.at[h],
                device_id=(right,), device_id_type=pl.DeviceIdType.MESH)
            r.start()
            return r

        ag = [ag_hop(0)]

        def issue_stage(t):
            j, qb = divmod(t, NQB)
            bb = batch_of(j)
            slot = t % 2
            ck = pltpu.make_async_copy(
                k_hbm.at[bb, :, qb, :, pl.ds(h0, HL), :], kst.at[slot],
                k_sems.at[slot])
            cv = pltpu.make_async_copy(
                v_hbm.at[bb, :, qb, :, pl.ds(h0, HL), :], vst.at[slot],
                v_sems.at[slot])
            ck.start()
            cv.start()
            return (ck, cv)

        desc = ({0: issue_stage(0), 1: issue_stage(1)}
                if ABLATE != "compute" else {})

        wq_bf[...] = wq_ref[...].astype(jnp.bfloat16)
        wo_bf[...] = wo_ref[...].astype(jnp.bfloat16)

        rs = [None] * (N_DEV - 1)

        def rs_step_start(s):
            c_send = lax.rem(my + (N_DEV - 1 - s), N_DEV)
            r = pltpu.make_async_remote_copy(
                src_ref=acc.at[pl.ds(c_send, 1)],
                dst_ref=rs_buf.at[pl.ds(s, 1)],
                send_sem=rs_send.at[s], recv_sem=rs_recv.at[s],
                device_id=(right,), device_id_type=pl.DeviceIdType.MESH)
            r.start()
            rs[s] = r

        def rs_step_finish(s):
            rs[s].wait()
            c_add = lax.rem(my + (N_DEV - 2 - s), N_DEV)
            acc[pl.ds(c_add, 1)] = acc[pl.ds(c_add, 1)] + rs_buf[pl.ds(s, 1)]

        def compute_batch(j):
            bb = batch_of(j)
            xb = x_all[pl.ds(bb, 1)][0]
            q = jnp.dot(xb, wq_bf[...],
                        preferred_element_type=jnp.float32)
            q_bf[...] = (q * SCALE).astype(jnp.bfloat16)

            for qb in range(NQB):
                t = j * NQB + qb
                slot = t % 2
                if ABLATE != "compute":
                    ck, cv = desc.pop(t)
                    ck.wait()
                    cv.wait()
                    k_bf[...] = kst[slot].reshape(KSEL, HL * DH
                                                  ).astype(jnp.bfloat16)
                    v_bf[...] = vst[slot].reshape(KSEL, HL * DH
                                                  ).astype(jnp.bfloat16)
                    if t + 2 < N_DEV * NQB:
                        desc[t + 2] = issue_stage(t + 2)

                for h in range(HL) if ABLATE != "mem" else ():
                    kh = k_bf[:, h * DH:(h + 1) * DH]
                    vh = v_bf[:, h * DH:(h + 1) * DH]
                    qh = q_bf[qb * QBLK:(qb + 1) * QBLK,
                              h * DH:(h + 1) * DH]
                    s = lax.dot_general(qh, kh, (((1,), (1,)), ((), ())),
                                        preferred_element_type=jnp.float32)
                    e = jnp.exp(s)
                    inv = 1.0 / jnp.sum(e, axis=-1, keepdims=True)
                    o = jnp.dot(e.astype(jnp.bfloat16), vh,
                                preferred_element_type=jnp.float32)
                    ctx_bf[:, h * DH:(h + 1) * DH] = (o * inv
                                                      ).astype(jnp.bfloat16)
                psum = jnp.dot(ctx_bf[...], wo_bf[...],
                               preferred_element_type=jnp.float32)
                acc[pl.ds(bb, 1), qb * QBLK:(qb + 1) * QBLK, :] = psum[None]

        compute_batch(0)

        for j in range(1, N_DEV):
            ag[j - 1].wait()
            if j < N_DEV - 1:
                ag.append(ag_hop(j))
            compute_batch(j)
            if j >= 2:
                rs_step_finish(j - 2)
            rs_step_start(j - 1)

        rs_step_finish(N_DEV - 2)
        out_ref[...] = acc[pl.ds(my, 1)]

    return pl.pallas_call(
        body,
        out_shape=jax.ShapeDtypeStruct((1, SQ, DM), jnp.float32),
        in_specs=[
            pl.BlockSpec(memory_space=pltpu.MemorySpace.VMEM),
            pl.BlockSpec(memory_space=pltpu.MemorySpace.VMEM),
            pl.BlockSpec(memory_space=pl.ANY),
            pl.BlockSpec(memory_space=pl.ANY),
            pl.BlockSpec(memory_space=pltpu.MemorySpace.VMEM),
        ],
        out_specs=pl.BlockSpec(memory_space=pltpu.MemorySpace.VMEM),
        scratch_shapes=[
            pltpu.VMEM((B, SQ, DM), jnp.bfloat16),
            pltpu.VMEM((DM, DM), jnp.bfloat16),
            pltpu.VMEM((DM, DM), jnp.bfloat16),
            pltpu.VMEM((SQ, HL * DH), jnp.bfloat16),
            pltpu.VMEM((2, NT, QBLK, HL, DH), jnp.float32),
            pltpu.VMEM((2, NT, QBLK, HL, DH), jnp.float32),
            pltpu.VMEM((KSEL, HL * DH), jnp.bfloat16),
            pltpu.VMEM((KSEL, HL * DH), jnp.bfloat16),
            pltpu.VMEM((QBLK, HL * DH), jnp.bfloat16),
            pltpu.VMEM((B, SQ, DM), jnp.float32),
            pltpu.VMEM((N_DEV - 1, SQ, DM), jnp.float32),
            pltpu.SemaphoreType.DMA((N_DEV - 1,)),
            pltpu.SemaphoreType.DMA((N_DEV - 1,)),
            pltpu.SemaphoreType.DMA((N_DEV - 1,)),
            pltpu.SemaphoreType.DMA((N_DEV - 1,)),
            pltpu.SemaphoreType.DMA((2,)),
            pltpu.SemaphoreType.DMA((2,)),
        ],
        compiler_params=pltpu.CompilerParams(
            collective_id=0, vmem_limit_bytes=64 * 1024 * 1024),
    )(x, Wq, K_r, V_r, Wo)


# device time: 66281 ns/iter; 3.1888x vs baseline; 1.0972x over previous
import os

import jax
import jax.numpy as jnp
from jax import lax
from jax.experimental import pallas as pl
from jax.experimental.pallas import tpu as pltpu

N_DEV = 4
B, SQ, DM = 4, 256, 1024
HG, HL, DH = 32, 8, 128
NQB, QBLK = 4, 64
NT = 16
KSEL = NT * QBLK
SCALE = 0.08838834764831843
ABLATE = os.environ.get("ABLATE", "")
_SKIP_DMA = ABLATE in ("compute", "comm")
_SKIP_MATH = ABLATE in ("mem", "comm")


def kernel(x, Wq, K_ext, V_ext, Wo):
    K_r = K_ext.reshape(B, NT, NQB, QBLK, HG, DH)
    V_r = V_ext.reshape(B, NT, NQB, QBLK, HG, DH)

    def body(x_ref, wq_ref, k_hbm, v_hbm, wo_ref, out_ref,
             x_all, wq_bf, wo_bf, q_bf, kst, vst, k_bf, v_bf, ctx_bf,
             acc, rs_buf,
             ag_send, ag_recv, rs_send, rs_recv, k_sems, v_sems):
        my = lax.axis_index("i")
        right = lax.rem(my + 1, N_DEV)
        left = lax.rem(my + N_DEV - 1, N_DEV)
        h0 = my * HL

        def batch_of(j):
            return lax.rem(my + (N_DEV - j), N_DEV)

        barrier = pltpu.get_barrier_semaphore()
        for nbr in (left, right):
            pl.semaphore_signal(barrier, inc=1, device_id=(nbr,),
                                device_id_type=pl.DeviceIdType.MESH)
        pl.semaphore_wait(barrier, 2)

        x_all[pl.ds(my, 1)] = x_ref[...].astype(jnp.bfloat16)

        def ag_hop(h):
            c = lax.rem(my + (N_DEV - h), N_DEV)
            r = pltpu.make_async_remote_copy(
                src_ref=x_all.at[pl.ds(c, 1)],
                dst_ref=x_all.at[pl.ds(c, 1)],
                send_sem=ag_send.at[h], recv_sem=ag_recv.at[h],
                device_id=(right,), device_id_type=pl.DeviceIdType.MESH)
            r.start()
            return r

        ag = [ag_hop(0)]

        def issue_stage(t):
            j, qb = divmod(t, NQB)
            bb = batch_of(j)
            slot = t % 2
            ck = pltpu.make_async_copy(
                k_hbm.at[bb, :, qb, :, pl.ds(h0, HL), :], kst.at[slot],
                k_sems.at[slot])
            cv = pltpu.make_async_copy(
                v_hbm.at[bb, :, qb, :, pl.ds(h0, HL), :], vst.at[slot],
                v_sems.at[slot])
            ck.start()
            cv.start()
            return (ck, cv)

        desc = {} if _SKIP_DMA else {0: issue_stage(0), 1: issue_stage(1)}

        wq_bf[...] = wq_ref[...].astype(jnp.bfloat16)
        wo_bf[...] = wo_ref[...].astype(jnp.bfloat16)

        rs = [None] * (N_DEV - 1)

        def rs_step_start(s):
            c_send = lax.rem(my + (N_DEV - 1 - s), N_DEV)
            r = pltpu.make_async_remote_copy(
                src_ref=acc.at[pl.ds(c_send, 1)],
                dst_ref=rs_buf.at[pl.ds(s, 1)],
                send_sem=rs_send.at[s], recv_sem=rs_recv.at[s],
                device_id=(right,), device_id_type=pl.DeviceIdType.MESH)
            r.start()
            rs[s] = r

        def rs_step_finish(s):
            rs[s].wait()
            c_add = lax.rem(my + (N_DEV - 2 - s), N_DEV)
            acc[pl.ds(c_add, 1)] = acc[pl.ds(c_add, 1)] + rs_buf[pl.ds(s, 1)]

        def compute_batch(j):
            bb = batch_of(j)
            xb = x_all[pl.ds(bb, 1)][0]
            q = jnp.dot(xb, wq_bf[...],
                        preferred_element_type=jnp.float32)
            q_bf[...] = (q * SCALE).astype(jnp.bfloat16)

            for qb in range(NQB):
                t = j * NQB + qb
                slot = t % 2
                if not _SKIP_DMA:
                    ck, cv = desc.pop(t)
                    ck.wait()
                    cv.wait()
                    k_bf[...] = kst[slot].reshape(KSEL, HL * DH
                                                  ).astype(jnp.bfloat16)
                    v_bf[...] = vst[slot].reshape(KSEL, HL * DH
                                                  ).astype(jnp.bfloat16)
                    if t + 2 < N_DEV * NQB:
                        desc[t + 2] = issue_stage(t + 2)

                for h in range(HL) if not _SKIP_MATH else ():
                    kh = k_bf[:, h * DH:(h + 1) * DH]
                    vh = v_bf[:, h * DH:(h + 1) * DH]
                    qh = q_bf[qb * QBLK:(qb + 1) * QBLK,
                              h * DH:(h + 1) * DH]
                    s = lax.dot_general(qh, kh, (((1,), (1,)), ((), ())),
                                        preferred_element_type=jnp.float32)
                    e = jnp.exp(s)
                    inv = 1.0 / jnp.sum(e, axis=-1, keepdims=True)
                    o = jnp.dot(e.astype(jnp.bfloat16), vh,
                                preferred_element_type=jnp.float32)
                    ctx_bf[:, h * DH:(h + 1) * DH] = (o * inv
                                                      ).astype(jnp.bfloat16)
                psum = jnp.dot(ctx_bf[...], wo_bf[...],
                               preferred_element_type=jnp.float32)
                acc[pl.ds(bb, 1), qb * QBLK:(qb + 1) * QBLK, :] = psum[None]

        compute_batch(0)

        for j in range(1, N_DEV):
            ag[j - 1].wait()
            if j < N_DEV - 1:
                ag.append(ag_hop(j))
            compute_batch(j)
            if j >= 2:
                rs_step_finish(j - 2)
            rs_step_start(j - 1)

        rs_step_finish(N_DEV - 2)
        out_ref[...] = acc[pl.ds(my, 1)]

    return pl.pallas_call(
        body,
        out_shape=jax.ShapeDtypeStruct((1, SQ, DM), jnp.float32),
        in_specs=[
            pl.BlockSpec(memory_space=pltpu.MemorySpace.VMEM),
            pl.BlockSpec(memory_space=pltpu.MemorySpace.VMEM),
            pl.BlockSpec(memory_space=pl.ANY),
            pl.BlockSpec(memory_space=pl.ANY),
            pl.BlockSpec(memory_space=pltpu.MemorySpace.VMEM),
        ],
        out_specs=pl.BlockSpec(memory_space=pltpu.MemorySpace.VMEM),
        scratch_shapes=[
            pltpu.VMEM((B, SQ, DM), jnp.bfloat16),
            pltpu.VMEM((DM, DM), jnp.bfloat16),
            pltpu.VMEM((DM, DM), jnp.bfloat16),
            pltpu.VMEM((SQ, HL * DH), jnp.bfloat16),
            pltpu.VMEM((2, NT, QBLK, HL, DH), jnp.float32),
            pltpu.VMEM((2, NT, QBLK, HL, DH), jnp.float32),
            pltpu.VMEM((KSEL, HL * DH), jnp.bfloat16),
            pltpu.VMEM((KSEL, HL * DH), jnp.bfloat16),
            pltpu.VMEM((QBLK, HL * DH), jnp.bfloat16),
            pltpu.VMEM((B, SQ, DM), jnp.float32),
            pltpu.VMEM((N_DEV - 1, SQ, DM), jnp.float32),
            pltpu.SemaphoreType.DMA((N_DEV - 1,)),
            pltpu.SemaphoreType.DMA((N_DEV - 1,)),
            pltpu.SemaphoreType.DMA((N_DEV - 1,)),
            pltpu.SemaphoreType.DMA((N_DEV - 1,)),
            pltpu.SemaphoreType.DMA((2,)),
            pltpu.SemaphoreType.DMA((2,)),
        ],
        compiler_params=pltpu.CompilerParams(
            collective_id=0, vmem_limit_bytes=64 * 1024 * 1024),
    )(x, Wq, K_r, V_r, Wo)
